# baseline (speedup 1.0000x reference)
from math import sqrt

import jax
import jax.numpy as jnp
from jax import lax
from jax.experimental import pallas as pl
from jax.experimental.pallas import tpu as pltpu

_CLUSTERS = 128
_ITERATIONS = 10
_BITS = 32


def _ca_kernel(w2_ref, bias2_ref, init_ref, q_ref, k_ref, v_ref, out_ref):
    q = q_ref[0]
    k = k_ref[0]
    v = v_ref[0]
    L = q.shape[0]
    C = _CLUSTERS
    C2 = 2 * C
    f32 = jnp.float32
    bf16 = jnp.bfloat16

    lane = lax.broadcasted_iota(jnp.int32, (C2, 128), 1)
    row = lax.broadcasted_iota(jnp.int32, (C2, 128), 0)
    head_b = (row >= C).astype(jnp.int32)
    off_bits = head_b * _BITS
    mask_bits = ((lane >= off_bits) & (lane < off_bits + _BITS)).astype(f32)
    off_e = head_b * 64
    mask_e = ((lane >= off_e) & (lane < off_e + 64)).astype(f32)
    cid = (lax.broadcasted_iota(jnp.int32, (C2, 1), 0) & (C - 1)).astype(f32)

    proj = lax.dot_general(q, w2_ref[...], (((1,), (0,)), ((), ())),
                           preferred_element_type=f32) + bias2_ref[...]
    bits = (proj > 0).astype(f32)
    ones_col = (lax.broadcasted_iota(jnp.int32, (L, 128), 1) == 2 * _BITS)
    b_aug = jnp.concatenate([bits, jnp.zeros((L, 64), f32)], axis=1)
    b_aug = jnp.where(ones_col, 1.0, b_aug)
    b_aug_bf = b_aug.astype(bf16)

    iota_pos = lax.broadcasted_iota(jnp.int32, (C, L), 1)
    sel = (iota_pos == init_ref[...]).astype(bf16)
    c_half = lax.dot_general(sel, b_aug_bf, (((1,), (0,)), ((), ())),
                             preferred_element_type=f32)
    c_bits0 = jnp.concatenate([c_half, c_half], axis=0)

    one_bf = jnp.ones((), bf16)
    zero_bf = jnp.zeros((), bf16)

    def assign_onehot(c_bits):
        cb = c_bits * mask_bits
        pc = jnp.sum(cb, axis=1, keepdims=True)
        alpha = pc * 128.0 + cid
        key = lax.dot_general((cb * -256.0).astype(bf16), b_aug_bf,
                              (((1,), (1,)), ((), ())),
                              preferred_element_type=f32) + alpha
        ka = key[:C]
        kb = key[C:]
        oa = (ka == jnp.min(ka, axis=0, keepdims=True)).astype(f32)
        ob = (kb == jnp.min(kb, axis=0, keepdims=True)).astype(f32)
        return jnp.concatenate([oa, ob], axis=0).astype(bf16)

    def body(_, carry):
        c_bits, _, _ = carry
        onehot = assign_onehot(c_bits)
        bs = lax.dot_general(onehot, b_aug_bf, (((1,), (0,)), ((), ())),
                             preferred_element_type=f32)
        counts = bs[:, 2 * _BITS:2 * _BITS + 1]
        maj = (bs * 2.0 > counts).astype(f32)
        new_bits = jnp.where(counts > 0, maj, c_bits)
        return (new_bits, onehot, counts)

    zero_oh = jnp.zeros((C2, L), bf16)
    zero_ct = jnp.zeros((C2, 1), f32)
    carry = lax.fori_loop(0, _ITERATIONS + 1, body,
                          (c_bits0, zero_oh, zero_ct))
    _, onehot_bf, counts = carry
    onehot = onehot_bf.astype(f32)

    counts_c = jnp.maximum(counts, 1.0)
    q_sum = lax.dot_general(onehot, q, (((1,), (0,)), ((), ())),
                            preferred_element_type=f32)
    qg = (q_sum / counts_c) * mask_e
    qk = lax.dot_general(qg, k, (((1,), (1,)), ((), ())),
                         preferred_element_type=f32)
    qk = qk * (1.0 / sqrt(64.0))
    qk = qk - jnp.max(qk, axis=1, keepdims=True)
    e = jnp.exp(qk)
    a = e / jnp.sum(e, axis=1, keepdims=True)
    vc = lax.dot_general(a, v, (((1,), (0,)), ((), ())),
                         preferred_element_type=f32)
    vcm = vc * mask_e

    out_ref[0] = lax.dot_general(onehot, vcm, (((0,), (0,)), ((), ())),
                                 preferred_element_type=f32)


def kernel(queries, keys, values):
    N, L, H, E = queries.shape
    D = values.shape[-1]
    NP = (H * E) // 128

    qf = queries.reshape(N, L, H * E)
    kf = keys.reshape(N, L, H * E)
    vf = values.reshape(N, L, H * D)

    planes = jax.random.normal(jax.random.key(42), (_BITS, E + 1),
                               dtype=jnp.float32)
    w = planes[:, :-1].T
    bias = planes[:, -1]
    w2 = jnp.zeros((2 * E, 2 * _BITS), jnp.float32)
    w2 = w2.at[:E, :_BITS].set(w).at[E:, _BITS:].set(w)
    bias2 = jnp.concatenate([bias, bias])[None, :]
    init_idx = jnp.linspace(0, L - 1, _CLUSTERS).astype(jnp.int32)[:, None]

    grid = (N, NP)
    out = pl.pallas_call(
        _ca_kernel,
        grid=grid,
        in_specs=[
            pl.BlockSpec((2 * E, 2 * _BITS), lambda n, p: (0, 0)),
            pl.BlockSpec((1, 2 * _BITS), lambda n, p: (0, 0)),
            pl.BlockSpec((_CLUSTERS, 1), lambda n, p: (0, 0)),
            pl.BlockSpec((1, L, 128), lambda n, p: (n, 0, p)),
            pl.BlockSpec((1, L, 128), lambda n, p: (n, 0, p)),
            pl.BlockSpec((1, L, 128), lambda n, p: (n, 0, p)),
        ],
        out_specs=pl.BlockSpec((1, L, 128), lambda n, p: (n, 0, p)),
        out_shape=jax.ShapeDtypeStruct((N, L, H * D), jnp.float32),
        compiler_params=pltpu.CompilerParams(
            dimension_semantics=("arbitrary", "arbitrary"),
        ),
    )(w2, bias2, init_idx, qf, kf, vf)

    return out.reshape(N, L, H, D)


# speedup vs baseline: 1.0495x; 1.0495x over previous
from math import sqrt

import jax
import jax.numpy as jnp
from jax import lax
from jax.experimental import pallas as pl
from jax.experimental.pallas import tpu as pltpu

_CLUSTERS = 128
_ITERATIONS = 10
_BITS = 32


def _ca_kernel(w2_ref, bias2_ref, init_ref, q_ref, k_ref, v_ref, out_ref):
    q = q_ref[0]
    k = k_ref[0]
    v = v_ref[0]
    L = q.shape[0]
    C = _CLUSTERS
    C2 = 2 * C
    f32 = jnp.float32
    bf16 = jnp.bfloat16

    lane = lax.broadcasted_iota(jnp.int32, (C2, 128), 1)
    row = lax.broadcasted_iota(jnp.int32, (C2, 128), 0)
    head_b = (row >= C).astype(jnp.int32)
    off_bits = head_b * _BITS
    mask_bits = ((lane >= off_bits) & (lane < off_bits + _BITS)).astype(f32)
    off_e = head_b * 64
    mask_e = ((lane >= off_e) & (lane < off_e + 64)).astype(f32)
    cid = (lax.broadcasted_iota(jnp.int32, (C2, 1), 0) & (C - 1)).astype(f32)

    proj = lax.dot_general(q, w2_ref[...], (((1,), (0,)), ((), ())),
                           preferred_element_type=f32) + bias2_ref[...]
    bits = (proj > 0).astype(f32)
    ones_col = (lax.broadcasted_iota(jnp.int32, (L, 128), 1) == 2 * _BITS)
    b_aug = jnp.concatenate([bits, jnp.zeros((L, 64), f32)], axis=1)
    b_aug = jnp.where(ones_col, 1.0, b_aug)

    iota_pos = lax.broadcasted_iota(jnp.int32, (C, L), 1)
    sel = (iota_pos == init_ref[...]).astype(f32)
    c_half = lax.dot_general(sel, b_aug, (((1,), (0,)), ((), ())),
                             preferred_element_type=f32)
    c_bits0 = jnp.concatenate([c_half, c_half], axis=0)

    def assign_onehot(c_bits):
        cb = c_bits * mask_bits
        pc = jnp.sum(cb, axis=1, keepdims=True)
        alpha = pc * 128.0 + cid
        cb = cb * -256.0
        cb = jnp.where(lane == 2 * _BITS, alpha, cb)
        key = lax.dot_general(cb, b_aug, (((1,), (1,)), ((), ())),
                              preferred_element_type=f32)
        key3 = key.reshape(2, C, L)
        m = jnp.min(key3, axis=1, keepdims=True)
        onehot = (key3 == m).astype(f32).reshape(C2, L)
        return onehot

    def body(_, carry):
        c_bits, _, _ = carry
        onehot = assign_onehot(c_bits)
        bs = lax.dot_general(onehot, b_aug, (((1,), (0,)), ((), ())),
                             preferred_element_type=f32)
        counts = bs[:, 2 * _BITS:2 * _BITS + 1]
        maj = (bs * 2.0 > counts).astype(f32)
        new_bits = jnp.where(counts > 0, maj, c_bits)
        return (new_bits, onehot, counts)

    zero_oh = jnp.zeros((C2, L), f32)
    zero_ct = jnp.zeros((C2, 1), f32)
    carry = lax.fori_loop(0, _ITERATIONS + 1, body,
                          (c_bits0, zero_oh, zero_ct))
    _, onehot, counts = carry

    counts_c = jnp.maximum(counts, 1.0)
    q_sum = lax.dot_general(onehot, q, (((1,), (0,)), ((), ())),
                            preferred_element_type=f32)
    qg = (q_sum / counts_c) * mask_e
    qk = lax.dot_general(qg, k, (((1,), (1,)), ((), ())),
                         preferred_element_type=f32)
    qk = qk * (1.0 / sqrt(64.0))
    qk = qk - jnp.max(qk, axis=1, keepdims=True)
    e = jnp.exp(qk)
    a = e / jnp.sum(e, axis=1, keepdims=True)
    vc = lax.dot_general(a, v, (((1,), (0,)), ((), ())),
                         preferred_element_type=f32)
    vcm = vc * mask_e

    out_ref[0] = lax.dot_general(onehot, vcm, (((0,), (0,)), ((), ())),
                                 preferred_element_type=f32)


def kernel(queries, keys, values):
    N, L, H, E = queries.shape
    D = values.shape[-1]
    NP = (H * E) // 128

    qf = queries.reshape(N, L, H * E)
    kf = keys.reshape(N, L, H * E)
    vf = values.reshape(N, L, H * D)

    planes = jax.random.normal(jax.random.key(42), (_BITS, E + 1),
                               dtype=jnp.float32)
    w = planes[:, :-1].T
    bias = planes[:, -1]
    w2 = jnp.zeros((2 * E, 2 * _BITS), jnp.float32)
    w2 = w2.at[:E, :_BITS].set(w).at[E:, _BITS:].set(w)
    bias2 = jnp.concatenate([bias, bias])[None, :]
    init_idx = jnp.linspace(0, L - 1, _CLUSTERS).astype(jnp.int32)[:, None]

    grid = (N, NP)
    out = pl.pallas_call(
        _ca_kernel,
        grid=grid,
        in_specs=[
            pl.BlockSpec((2 * E, 2 * _BITS), lambda n, p: (0, 0)),
            pl.BlockSpec((1, 2 * _BITS), lambda n, p: (0, 0)),
            pl.BlockSpec((_CLUSTERS, 1), lambda n, p: (0, 0)),
            pl.BlockSpec((1, L, 128), lambda n, p: (n, 0, p)),
            pl.BlockSpec((1, L, 128), lambda n, p: (n, 0, p)),
            pl.BlockSpec((1, L, 128), lambda n, p: (n, 0, p)),
        ],
        out_specs=pl.BlockSpec((1, L, 128), lambda n, p: (n, 0, p)),
        out_shape=jax.ShapeDtypeStruct((N, L, H * D), jnp.float32),
        compiler_params=pltpu.CompilerParams(
            dimension_semantics=("arbitrary", "arbitrary"),
        ),
    )(w2, bias2, init_idx, qf, kf, vf)

    return out.reshape(N, L, H, D)


# speedup vs baseline: 1.0778x; 1.0269x over previous
from math import sqrt

import jax
import jax.numpy as jnp
from jax import lax
from jax.experimental import pallas as pl
from jax.experimental.pallas import tpu as pltpu

_CLUSTERS = 128
_ITERATIONS = 10
_BITS = 32


def _ca_kernel(w2_ref, bias2_ref, init_ref, q_ref, k_ref, v_ref, out_ref):
    q = q_ref[0]
    k = k_ref[0]
    v = v_ref[0]
    L = q.shape[0]
    C = _CLUSTERS
    f32 = jnp.float32

    lane_c = lax.broadcasted_iota(jnp.int32, (C, 128), 1)
    mask_bits_a = (lane_c < _BITS).astype(f32)
    mask_bits_b = ((lane_c >= _BITS) & (lane_c < 2 * _BITS)).astype(f32)
    mask_e_a = (lane_c < 64).astype(f32)
    mask_e_b = (lane_c >= 64).astype(f32)
    iota_c1 = lax.broadcasted_iota(jnp.int32, (C, 1), 0).astype(f32)

    proj = lax.dot_general(q, w2_ref[...], (((1,), (0,)), ((), ())),
                           preferred_element_type=f32) + bias2_ref[...]
    bits = (proj > 0).astype(f32)
    ones_col = (lax.broadcasted_iota(jnp.int32, (L, 128), 1) == 2 * _BITS)
    b_aug = jnp.concatenate([bits, jnp.zeros((L, 64), f32)], axis=1)
    b_aug = jnp.where(ones_col, 1.0, b_aug)
    b_aug_bf = b_aug.astype(jnp.bfloat16)

    iota_pos = lax.broadcasted_iota(jnp.int32, (C, L), 1)
    sel = (iota_pos == init_ref[...]).astype(f32).astype(jnp.bfloat16)
    c_bits0 = lax.dot_general(sel, b_aug_bf, (((1,), (0,)), ((), ())),
                              preferred_element_type=f32)

    def assign_onehot(c_bits, mask_bits):
        cb = c_bits * mask_bits
        pc = jnp.sum(cb, axis=1, keepdims=True)
        scores = lax.dot_general(cb, b_aug, (((1,), (1,)), ((), ())),
                                 preferred_element_type=f32)
        key = (pc * 128.0 + iota_c1) - 256.0 * scores
        m = jnp.min(key, axis=0, keepdims=True)
        onehot = (key == m).astype(f32)
        return onehot

    def body(_, carry):
        c_bits, _, _, _, _ = carry
        onehot_a = assign_onehot(c_bits, mask_bits_a)
        onehot_b = assign_onehot(c_bits, mask_bits_b)
        bs_a = lax.dot_general(onehot_a.astype(jnp.bfloat16), b_aug_bf,
                               (((1,), (0,)), ((), ())),
                               preferred_element_type=f32)
        bs_b = lax.dot_general(onehot_b.astype(jnp.bfloat16), b_aug_bf,
                               (((1,), (0,)), ((), ())),
                               preferred_element_type=f32)
        counts_a = bs_a[:, 2 * _BITS:2 * _BITS + 1]
        counts_b = bs_b[:, 2 * _BITS:2 * _BITS + 1]
        maj_a = (bs_a * 2.0 > counts_a).astype(f32)
        maj_b = (bs_b * 2.0 > counts_b).astype(f32)
        upd_a = jnp.where(counts_a > 0, maj_a, c_bits)
        upd_b = jnp.where(counts_b > 0, maj_b, c_bits)
        new_bits = jnp.where(lane_c < _BITS, upd_a, upd_b)
        return (new_bits, onehot_a, onehot_b, counts_a, counts_b)

    zero_oh = jnp.zeros((C, L), f32)
    zero_ct = jnp.zeros((C, 1), f32)
    carry = lax.fori_loop(0, _ITERATIONS + 1, body,
                          (c_bits0, zero_oh, zero_oh, zero_ct, zero_ct))
    _, onehot_a, onehot_b, counts_a, counts_b = carry

    temp = 1.0 / sqrt(64.0)

    def head_attention(onehot, counts, mask_e):
        counts_c = jnp.maximum(counts, 1.0)
        q_sum = lax.dot_general(onehot, q, (((1,), (0,)), ((), ())),
                                preferred_element_type=f32)
        qg = (q_sum / counts_c) * mask_e
        qk = lax.dot_general(qg, k, (((1,), (1,)), ((), ())),
                             preferred_element_type=f32)
        qk = qk * temp
        qk = qk - jnp.max(qk, axis=1, keepdims=True)
        e = jnp.exp(qk)
        a = e / jnp.sum(e, axis=1, keepdims=True)
        vc = lax.dot_general(a, v, (((1,), (0,)), ((), ())),
                             preferred_element_type=f32)
        return vc * mask_e

    vc_a = head_attention(onehot_a, counts_a, mask_e_a)
    vc_b = head_attention(onehot_b, counts_b, mask_e_b)

    out_a = lax.dot_general(onehot_a, vc_a, (((0,), (0,)), ((), ())),
                            preferred_element_type=f32)
    out_b = lax.dot_general(onehot_b, vc_b, (((0,), (0,)), ((), ())),
                            preferred_element_type=f32)
    out_ref[0] = out_a + out_b


def kernel(queries, keys, values):
    N, L, H, E = queries.shape
    D = values.shape[-1]
    NP = (H * E) // 128

    qf = queries.reshape(N, L, H * E)
    kf = keys.reshape(N, L, H * E)
    vf = values.reshape(N, L, H * D)

    planes = jax.random.normal(jax.random.key(42), (_BITS, E + 1),
                               dtype=jnp.float32)
    w = planes[:, :-1].T
    bias = planes[:, -1]
    w2 = jnp.zeros((2 * E, 2 * _BITS), jnp.float32)
    w2 = w2.at[:E, :_BITS].set(w).at[E:, _BITS:].set(w)
    bias2 = jnp.concatenate([bias, bias])[None, :]
    init_idx = jnp.linspace(0, L - 1, _CLUSTERS).astype(jnp.int32)[:, None]

    grid = (N, NP)
    out = pl.pallas_call(
        _ca_kernel,
        grid=grid,
        in_specs=[
            pl.BlockSpec((2 * E, 2 * _BITS), lambda n, p: (0, 0)),
            pl.BlockSpec((1, 2 * _BITS), lambda n, p: (0, 0)),
            pl.BlockSpec((_CLUSTERS, 1), lambda n, p: (0, 0)),
            pl.BlockSpec((1, L, 128), lambda n, p: (n, 0, p)),
            pl.BlockSpec((1, L, 128), lambda n, p: (n, 0, p)),
            pl.BlockSpec((1, L, 128), lambda n, p: (n, 0, p)),
        ],
        out_specs=pl.BlockSpec((1, L, 128), lambda n, p: (n, 0, p)),
        out_shape=jax.ShapeDtypeStruct((N, L, H * D), jnp.float32),
        compiler_params=pltpu.CompilerParams(
            dimension_semantics=("parallel", "parallel"),
        ),
    )(w2, bias2, init_idx, qf, kf, vf)

    return out.reshape(N, L, H, D)


# speedup vs baseline: 1.0976x; 1.0183x over previous
from math import sqrt

import jax
import jax.numpy as jnp
from jax import lax
from jax.experimental import pallas as pl
from jax.experimental.pallas import tpu as pltpu

_CLUSTERS = 128
_ITERATIONS = 10
_BITS = 32


def _ca_kernel(w2_ref, bias2_ref, init_ref, q_ref, k_ref, v_ref, out_ref):
    q = q_ref[0]
    k = k_ref[0]
    v = v_ref[0]
    L = q.shape[0]
    C = _CLUSTERS
    f32 = jnp.float32

    lane_c = lax.broadcasted_iota(jnp.int32, (C, 128), 1)
    mask_bits_a = (lane_c < _BITS).astype(f32)
    mask_bits_b = ((lane_c >= _BITS) & (lane_c < 2 * _BITS)).astype(f32)
    mask_e_a = (lane_c < 64).astype(f32)
    mask_e_b = (lane_c >= 64).astype(f32)
    iota_c1 = lax.broadcasted_iota(jnp.int32, (C, 1), 0).astype(f32)

    proj = lax.dot_general(q, w2_ref[...], (((1,), (0,)), ((), ())),
                           preferred_element_type=f32) + bias2_ref[...]
    bits = (proj > 0).astype(f32)
    ones_col = (lax.broadcasted_iota(jnp.int32, (L, 128), 1) == 2 * _BITS)
    b_aug = jnp.concatenate([bits, jnp.zeros((L, 64), f32)], axis=1)
    b_aug = jnp.where(ones_col, 1.0, b_aug)
    b_aug_bf = b_aug.astype(jnp.bfloat16)

    iota_pos = lax.broadcasted_iota(jnp.int32, (C, L), 1)
    sel = (iota_pos == init_ref[...]).astype(f32)
    c_bits0 = lax.dot_general(sel, b_aug, (((1,), (0,)), ((), ())),
                              preferred_element_type=f32)

    def assign_onehot(c_bits, mask_bits):
        cb = c_bits * mask_bits
        pc = jnp.sum(cb, axis=1, keepdims=True)
        scores = lax.dot_general(cb.astype(jnp.bfloat16), b_aug_bf,
                                 (((1,), (1,)), ((), ())),
                                 preferred_element_type=f32)
        key = (pc * 128.0 + iota_c1) - 256.0 * scores
        m = jnp.min(key, axis=0, keepdims=True)
        onehot = (key == m).astype(f32)
        return onehot

    def body(_, carry):
        c_bits, _, _, _, _ = carry
        onehot_a = assign_onehot(c_bits, mask_bits_a)
        onehot_b = assign_onehot(c_bits, mask_bits_b)
        bs_a = lax.dot_general(onehot_a, b_aug, (((1,), (0,)), ((), ())),
                               preferred_element_type=f32)
        bs_b = lax.dot_general(onehot_b, b_aug, (((1,), (0,)), ((), ())),
                               preferred_element_type=f32)
        counts_a = bs_a[:, 2 * _BITS:2 * _BITS + 1]
        counts_b = bs_b[:, 2 * _BITS:2 * _BITS + 1]
        maj_a = (bs_a * 2.0 > counts_a).astype(f32)
        maj_b = (bs_b * 2.0 > counts_b).astype(f32)
        upd_a = jnp.where(counts_a > 0, maj_a, c_bits)
        upd_b = jnp.where(counts_b > 0, maj_b, c_bits)
        new_bits = jnp.where(lane_c < _BITS, upd_a, upd_b)
        return (new_bits, onehot_a, onehot_b, counts_a, counts_b)

    zero_oh = jnp.zeros((C, L), f32)
    zero_ct = jnp.zeros((C, 1), f32)
    carry = lax.fori_loop(0, _ITERATIONS + 1, body,
                          (c_bits0, zero_oh, zero_oh, zero_ct, zero_ct))
    _, onehot_a, onehot_b, counts_a, counts_b = carry

    temp = 1.0 / sqrt(64.0)

    def head_attention(onehot, counts, mask_e):
        counts_c = jnp.maximum(counts, 1.0)
        q_sum = lax.dot_general(onehot, q, (((1,), (0,)), ((), ())),
                                preferred_element_type=f32)
        qg = (q_sum / counts_c) * mask_e
        qk = lax.dot_general(qg, k, (((1,), (1,)), ((), ())),
                             preferred_element_type=f32)
        qk = qk * temp
        qk = qk - jnp.max(qk, axis=1, keepdims=True)
        e = jnp.exp(qk)
        a = e / jnp.sum(e, axis=1, keepdims=True)
        vc = lax.dot_general(a, v, (((1,), (0,)), ((), ())),
                             preferred_element_type=f32)
        return vc * mask_e

    vc_a = head_attention(onehot_a, counts_a, mask_e_a)
    vc_b = head_attention(onehot_b, counts_b, mask_e_b)

    out_a = lax.dot_general(onehot_a, vc_a, (((0,), (0,)), ((), ())),
                            preferred_element_type=f32)
    out_b = lax.dot_general(onehot_b, vc_b, (((0,), (0,)), ((), ())),
                            preferred_element_type=f32)
    out_ref[0] = out_a + out_b


def kernel(queries, keys, values):
    N, L, H, E = queries.shape
    D = values.shape[-1]
    NP = (H * E) // 128

    qf = queries.reshape(N, L, H * E)
    kf = keys.reshape(N, L, H * E)
    vf = values.reshape(N, L, H * D)

    planes = jax.random.normal(jax.random.key(42), (_BITS, E + 1),
                               dtype=jnp.float32)
    w = planes[:, :-1].T
    bias = planes[:, -1]
    w2 = jnp.zeros((2 * E, 2 * _BITS), jnp.float32)
    w2 = w2.at[:E, :_BITS].set(w).at[E:, _BITS:].set(w)
    bias2 = jnp.concatenate([bias, bias])[None, :]
    init_idx = jnp.linspace(0, L - 1, _CLUSTERS).astype(jnp.int32)[:, None]

    grid = (N, NP)
    out = pl.pallas_call(
        _ca_kernel,
        grid=grid,
        in_specs=[
            pl.BlockSpec((2 * E, 2 * _BITS), lambda n, p: (0, 0)),
            pl.BlockSpec((1, 2 * _BITS), lambda n, p: (0, 0)),
            pl.BlockSpec((_CLUSTERS, 1), lambda n, p: (0, 0)),
            pl.BlockSpec((1, L, 128), lambda n, p: (n, 0, p)),
            pl.BlockSpec((1, L, 128), lambda n, p: (n, 0, p)),
            pl.BlockSpec((1, L, 128), lambda n, p: (n, 0, p)),
        ],
        out_specs=pl.BlockSpec((1, L, 128), lambda n, p: (n, 0, p)),
        out_shape=jax.ShapeDtypeStruct((N, L, H * D), jnp.float32),
        compiler_params=pltpu.CompilerParams(
            dimension_semantics=("arbitrary", "arbitrary"),
        ),
    )(w2, bias2, init_idx, qf, kf, vf)

    return out.reshape(N, L, H, D)


# speedup vs baseline: 1.1079x; 1.0095x over previous
from math import sqrt

import jax
import jax.numpy as jnp
from jax import lax
from jax.experimental import pallas as pl
from jax.experimental.pallas import tpu as pltpu

_CLUSTERS = 128
_ITERATIONS = 10
_BITS = 32


def _ca_kernel(w2_ref, bias2_ref, init_ref, q_ref, k_ref, v_ref, out_ref):
    q = q_ref[0]
    k = k_ref[0]
    v = v_ref[0]
    L = q.shape[0]
    C = _CLUSTERS
    f32 = jnp.float32

    lane_c = lax.broadcasted_iota(jnp.int32, (C, 128), 1)
    mask_bits_a = (lane_c < _BITS).astype(f32)
    mask_bits_b = ((lane_c >= _BITS) & (lane_c < 2 * _BITS)).astype(f32)
    mask_e_a = (lane_c < 64).astype(f32)
    mask_e_b = (lane_c >= 64).astype(f32)
    iota_c1 = lax.broadcasted_iota(jnp.int32, (C, 1), 0).astype(f32)

    proj = lax.dot_general(q, w2_ref[...], (((1,), (0,)), ((), ())),
                           preferred_element_type=f32) + bias2_ref[...]
    bits = (proj > 0).astype(f32)
    ones_col = (lax.broadcasted_iota(jnp.int32, (L, 128), 1) == 2 * _BITS)
    b_aug = jnp.concatenate([bits, jnp.zeros((L, 64), f32)], axis=1)
    b_aug = jnp.where(ones_col, 1.0, b_aug)
    b_aug_bf = b_aug.astype(jnp.bfloat16)

    iota_pos = lax.broadcasted_iota(jnp.int32, (C, L), 1)
    sel = (iota_pos == init_ref[...]).astype(f32)
    c_bits0 = lax.dot_general(sel, b_aug, (((1,), (0,)), ((), ())),
                              preferred_element_type=f32)

    def assign_onehot(c_bits, mask_bits):
        cb = c_bits * mask_bits
        pc = jnp.sum(cb, axis=1, keepdims=True)
        alpha = pc * 128.0 + iota_c1
        cb2 = jnp.where(lane_c == 2 * _BITS, alpha, cb * -256.0)
        key = lax.dot_general(cb2, b_aug, (((1,), (1,)), ((), ())),
                              preferred_element_type=f32)
        m = jnp.min(key, axis=0, keepdims=True)
        onehot = (key == m).astype(f32)
        return onehot

    def body(_, carry):
        c_bits, _, _, _, _ = carry
        onehot_a = assign_onehot(c_bits, mask_bits_a)
        onehot_b = assign_onehot(c_bits, mask_bits_b)
        bs_a = lax.dot_general(onehot_a, b_aug, (((1,), (0,)), ((), ())),
                               preferred_element_type=f32)
        bs_b = lax.dot_general(onehot_b, b_aug, (((1,), (0,)), ((), ())),
                               preferred_element_type=f32)
        counts_a = bs_a[:, 2 * _BITS:2 * _BITS + 1]
        counts_b = bs_b[:, 2 * _BITS:2 * _BITS + 1]
        maj_a = (bs_a * 2.0 > counts_a).astype(f32)
        maj_b = (bs_b * 2.0 > counts_b).astype(f32)
        upd_a = jnp.where(counts_a > 0, maj_a, c_bits)
        upd_b = jnp.where(counts_b > 0, maj_b, c_bits)
        new_bits = jnp.where(lane_c < _BITS, upd_a, upd_b)
        return (new_bits, onehot_a, onehot_b, counts_a, counts_b)

    zero_oh = jnp.zeros((C, L), f32)
    zero_ct = jnp.zeros((C, 1), f32)
    carry = lax.fori_loop(0, _ITERATIONS + 1, body,
                          (c_bits0, zero_oh, zero_oh, zero_ct, zero_ct))
    _, onehot_a, onehot_b, counts_a, counts_b = carry

    temp = 1.0 / sqrt(64.0)

    def head_attention(onehot, counts, mask_e):
        counts_c = jnp.maximum(counts, 1.0)
        q_sum = lax.dot_general(onehot, q, (((1,), (0,)), ((), ())),
                                preferred_element_type=f32)
        qg = (q_sum / counts_c) * mask_e
        qk = lax.dot_general(qg, k, (((1,), (1,)), ((), ())),
                             preferred_element_type=f32)
        qk = qk * temp
        qk = qk - jnp.max(qk, axis=1, keepdims=True)
        e = jnp.exp(qk)
        a = e / jnp.sum(e, axis=1, keepdims=True)
        vc = lax.dot_general(a, v, (((1,), (0,)), ((), ())),
                             preferred_element_type=f32)
        return vc * mask_e

    vc_a = head_attention(onehot_a, counts_a, mask_e_a)
    vc_b = head_attention(onehot_b, counts_b, mask_e_b)

    out_a = lax.dot_general(onehot_a, vc_a, (((0,), (0,)), ((), ())),
                            preferred_element_type=f32)
    out_b = lax.dot_general(onehot_b, vc_b, (((0,), (0,)), ((), ())),
                            preferred_element_type=f32)
    out_ref[0] = out_a + out_b


def kernel(queries, keys, values):
    N, L, H, E = queries.shape
    D = values.shape[-1]
    NP = (H * E) // 128

    qf = queries.reshape(N, L, H * E)
    kf = keys.reshape(N, L, H * E)
    vf = values.reshape(N, L, H * D)

    planes = jax.random.normal(jax.random.key(42), (_BITS, E + 1),
                               dtype=jnp.float32)
    w = planes[:, :-1].T
    bias = planes[:, -1]
    w2 = jnp.zeros((2 * E, 2 * _BITS), jnp.float32)
    w2 = w2.at[:E, :_BITS].set(w).at[E:, _BITS:].set(w)
    bias2 = jnp.concatenate([bias, bias])[None, :]
    init_idx = jnp.linspace(0, L - 1, _CLUSTERS).astype(jnp.int32)[:, None]

    grid = (N, NP)
    out = pl.pallas_call(
        _ca_kernel,
        grid=grid,
        in_specs=[
            pl.BlockSpec((2 * E, 2 * _BITS), lambda n, p: (0, 0)),
            pl.BlockSpec((1, 2 * _BITS), lambda n, p: (0, 0)),
            pl.BlockSpec((_CLUSTERS, 1), lambda n, p: (0, 0)),
            pl.BlockSpec((1, L, 128), lambda n, p: (n, 0, p)),
            pl.BlockSpec((1, L, 128), lambda n, p: (n, 0, p)),
            pl.BlockSpec((1, L, 128), lambda n, p: (n, 0, p)),
        ],
        out_specs=pl.BlockSpec((1, L, 128), lambda n, p: (n, 0, p)),
        out_shape=jax.ShapeDtypeStruct((N, L, H * D), jnp.float32),
        compiler_params=pltpu.CompilerParams(
            dimension_semantics=("arbitrary", "arbitrary"),
        ),
    )(w2, bias2, init_idx, qf, kf, vf)

    return out.reshape(N, L, H, D)


# speedup vs baseline: 1.1104x; 1.0022x over previous
from math import sqrt

import jax
import jax.numpy as jnp
from jax import lax
from jax.experimental import pallas as pl
from jax.experimental.pallas import tpu as pltpu

_CLUSTERS = 128
_ITERATIONS = 10
_BITS = 32


def _ca_kernel(w2_ref, bias2_ref, init_ref, q_ref, k_ref, v_ref, out_ref):
    q = q_ref[0]
    k = k_ref[0]
    v = v_ref[0]
    L = q.shape[0]
    C = _CLUSTERS
    f32 = jnp.float32

    lane_c = lax.broadcasted_iota(jnp.int32, (C, 128), 1)
    mask_bits_a = (lane_c < _BITS).astype(f32)
    mask_bits_b = ((lane_c >= _BITS) & (lane_c < 2 * _BITS)).astype(f32)
    mask_e_a = (lane_c < 64).astype(f32)
    mask_e_b = (lane_c >= 64).astype(f32)
    iota_c1 = lax.broadcasted_iota(jnp.int32, (C, 1), 0).astype(f32)
    m64 = (lane_c == 2 * _BITS).astype(f32)
    m65 = (lane_c == 2 * _BITS + 1).astype(f32)

    proj = lax.dot_general(q, w2_ref[...], (((1,), (0,)), ((), ())),
                           preferred_element_type=f32) + bias2_ref[...]
    bits = (proj > 0).astype(f32)
    lane_l = lax.broadcasted_iota(jnp.int32, (L, 128), 1)
    ones_col = (lane_l == 2 * _BITS) | (lane_l == 2 * _BITS + 1)
    b_aug = jnp.concatenate([bits, jnp.zeros((L, 64), f32)], axis=1)
    b_aug = jnp.where(ones_col, 1.0, b_aug)
    b_aug_bf = b_aug.astype(jnp.bfloat16)

    iota_pos = lax.broadcasted_iota(jnp.int32, (C, L), 1)
    sel = (iota_pos == init_ref[...]).astype(f32)
    c_bits0 = lax.dot_general(sel, b_aug, (((1,), (0,)), ((), ())),
                              preferred_element_type=f32)

    def assign_onehot(c_bits, mask_bits):
        cb = c_bits * mask_bits
        pc = jnp.sum(cb, axis=1, keepdims=True)
        cb2 = cb * -256.0 + (pc * 128.0) * m64 + iota_c1 * m65
        key = lax.dot_general(cb2.astype(jnp.bfloat16), b_aug_bf,
                              (((1,), (1,)), ((), ())),
                              preferred_element_type=f32)
        m = jnp.min(key, axis=0, keepdims=True)
        onehot = (key == m).astype(f32)
        return onehot

    def body(_, carry):
        c_bits, _, _, _, _ = carry
        onehot_a = assign_onehot(c_bits, mask_bits_a)
        onehot_b = assign_onehot(c_bits, mask_bits_b)
        bs_a = lax.dot_general(onehot_a, b_aug, (((1,), (0,)), ((), ())),
                               preferred_element_type=f32)
        bs_b = lax.dot_general(onehot_b, b_aug, (((1,), (0,)), ((), ())),
                               preferred_element_type=f32)
        counts_a = bs_a[:, 2 * _BITS:2 * _BITS + 1]
        counts_b = bs_b[:, 2 * _BITS:2 * _BITS + 1]
        maj_a = (bs_a * 2.0 > counts_a).astype(f32)
        maj_b = (bs_b * 2.0 > counts_b).astype(f32)
        upd_a = jnp.where(counts_a > 0, maj_a, c_bits)
        upd_b = jnp.where(counts_b > 0, maj_b, c_bits)
        new_bits = jnp.where(lane_c < _BITS, upd_a, upd_b)
        return (new_bits, onehot_a, onehot_b, counts_a, counts_b)

    zero_oh = jnp.zeros((C, L), f32)
    zero_ct = jnp.zeros((C, 1), f32)
    carry = lax.fori_loop(0, _ITERATIONS + 1, body,
                          (c_bits0, zero_oh, zero_oh, zero_ct, zero_ct))
    _, onehot_a, onehot_b, counts_a, counts_b = carry

    temp = 1.0 / sqrt(64.0)

    def head_attention(onehot, counts, mask_e):
        counts_c = jnp.maximum(counts, 1.0)
        q_sum = lax.dot_general(onehot, q, (((1,), (0,)), ((), ())),
                                preferred_element_type=f32)
        qg = (q_sum / counts_c) * mask_e
        qk = lax.dot_general(qg, k, (((1,), (1,)), ((), ())),
                             preferred_element_type=f32)
        qk = qk * temp
        qk = qk - jnp.max(qk, axis=1, keepdims=True)
        e = jnp.exp(qk)
        a = e / jnp.sum(e, axis=1, keepdims=True)
        vc = lax.dot_general(a, v, (((1,), (0,)), ((), ())),
                             preferred_element_type=f32)
        return vc * mask_e

    vc_a = head_attention(onehot_a, counts_a, mask_e_a)
    vc_b = head_attention(onehot_b, counts_b, mask_e_b)

    out_a = lax.dot_general(onehot_a, vc_a, (((0,), (0,)), ((), ())),
                            preferred_element_type=f32)
    out_b = lax.dot_general(onehot_b, vc_b, (((0,), (0,)), ((), ())),
                            preferred_element_type=f32)
    out_ref[0] = out_a + out_b


def kernel(queries, keys, values):
    N, L, H, E = queries.shape
    D = values.shape[-1]
    NP = (H * E) // 128

    qf = queries.reshape(N, L, H * E)
    kf = keys.reshape(N, L, H * E)
    vf = values.reshape(N, L, H * D)

    planes = jax.random.normal(jax.random.key(42), (_BITS, E + 1),
                               dtype=jnp.float32)
    w = planes[:, :-1].T
    bias = planes[:, -1]
    w2 = jnp.zeros((2 * E, 2 * _BITS), jnp.float32)
    w2 = w2.at[:E, :_BITS].set(w).at[E:, _BITS:].set(w)
    bias2 = jnp.concatenate([bias, bias])[None, :]
    init_idx = jnp.linspace(0, L - 1, _CLUSTERS).astype(jnp.int32)[:, None]

    grid = (N, NP)
    out = pl.pallas_call(
        _ca_kernel,
        grid=grid,
        in_specs=[
            pl.BlockSpec((2 * E, 2 * _BITS), lambda n, p: (0, 0)),
            pl.BlockSpec((1, 2 * _BITS), lambda n, p: (0, 0)),
            pl.BlockSpec((_CLUSTERS, 1), lambda n, p: (0, 0)),
            pl.BlockSpec((1, L, 128), lambda n, p: (n, 0, p)),
            pl.BlockSpec((1, L, 128), lambda n, p: (n, 0, p)),
            pl.BlockSpec((1, L, 128), lambda n, p: (n, 0, p)),
        ],
        out_specs=pl.BlockSpec((1, L, 128), lambda n, p: (n, 0, p)),
        out_shape=jax.ShapeDtypeStruct((N, L, H * D), jnp.float32),
        compiler_params=pltpu.CompilerParams(
            dimension_semantics=("arbitrary", "arbitrary"),
        ),
    )(w2, bias2, init_idx, qf, kf, vf)

    return out.reshape(N, L, H, D)
